# baseline (speedup 1.0000x reference)
import functools

import jax
import jax.numpy as jnp
from jax import lax
from jax.experimental import pallas as pl
from jax.experimental.pallas import tpu as pltpu
from jax.experimental.pallas import tpu_sc as plsc

_S = 1024
_K = 32
_SB = 128
_SBLK = 128
_PAD = 128


def _fps_body(xr_ref, out_ref):
    xyz24 = xr_ref[0].reshape(24, 1024)
    rows = lax.broadcasted_iota(jnp.int32, (8, 1024), 0)
    cols = lax.broadcasted_iota(jnp.int32, (8, 1024), 1)
    lin = rows * 1024 + cols
    lin24 = jnp.concatenate([lin, lin, lin], axis=0)
    rows_s = lax.broadcasted_iota(jnp.int32, (8, 128), 0)
    cols_s = lax.broadcasted_iota(jnp.int32, (8, 128), 1)
    lin_s = rows_s * 128 + cols_s

    def body(t, carry):
        last, dist, acc = carry
        ps = jnp.sum(jnp.where(lin24 == last, xyz24, 0.0), axis=1, keepdims=True)
        p = jnp.broadcast_to(
            jnp.sum(ps.reshape(3, 8, 1), axis=1, keepdims=True), (3, 8, 1)
        ).reshape(24, 1)
        d24 = (xyz24 - p) ** 2
        d = (d24[0:8] + d24[8:16]) + d24[16:24]
        dist = jnp.minimum(dist, d)
        m = jnp.max(jnp.max(dist, axis=1, keepdims=True), axis=0, keepdims=True)
        cand = jnp.where(dist == m, lin, jnp.int32(2 ** 30))
        nxt = jnp.min(jnp.min(cand, axis=1, keepdims=True), axis=0, keepdims=True)
        acc = jnp.where(lin_s == t, nxt, acc)
        return nxt, dist, acc

    dist0 = jnp.full((8, 1024), 1e10, jnp.float32)
    acc0 = jnp.zeros((8, 128), jnp.int32)
    _, _, acc = lax.fori_loop(
        1, _S, body, (jnp.zeros((1, 1), jnp.int32), dist0, acc0))
    out_ref[0] = acc


def _knn_body(xt_ref, q_ref, out_ref, d2_ref):
    xm = xt_ref[0]
    q = q_ref[0]
    n = xm.shape[-1]
    xsq = jnp.sum(xm * xm, axis=0, keepdims=True)
    qsq = jnp.sum(q * q, axis=1, keepdims=True)
    prod = lax.dot_general(q, xm, (((1,), (0,)), ((), ())),
                           preferred_element_type=jnp.float32)
    d2_ref[...] = (qsq - 2.0 * prod) + xsq
    lane = lax.broadcasted_iota(jnp.int32, (_SB, n), 1)
    kcol = lax.broadcasted_iota(jnp.int32, (_SB, _K), 1)
    big = jnp.float32(3.0e38)

    def body(t, _):
        d2 = d2_ref[...]
        m = jnp.min(d2, axis=1, keepdims=True)
        nxt = jnp.min(jnp.where(d2 == m, lane, jnp.int32(2 ** 30)),
                      axis=1, keepdims=True)
        out_ref[0] = jnp.where(kcol == t, nxt, out_ref[0])
        d2_ref[...] = jnp.where(lane == nxt, big, d2)
        return 0

    lax.fori_loop(0, _K, body, 0)


def _make_sc_gather(n_rows, width):
    info = plsc.get_sparse_core_info()
    nc, ns = info.num_cores, info.num_subcores
    nw = nc * ns
    per_w = n_rows // nw
    chunk = 128
    n_chunks = per_w // chunk
    mesh = plsc.VectorSubcoreMesh(core_axis_name="c", subcore_axis_name="s")

    @functools.partial(
        pl.kernel, mesh=mesh,
        out_type=jax.ShapeDtypeStruct((n_rows, width), jnp.float32),
        scratch_types=[
            pltpu.VMEM((chunk,), jnp.int32),
            pltpu.VMEM((chunk, width), jnp.float32),
            pltpu.SemaphoreType.DMA,
        ],
    )
    def gather(tbl_hbm, idx_hbm, out_hbm, idx_v, rows_v, sem):
        wid = lax.axis_index("s") * nc + lax.axis_index("c")
        w_base = wid * per_w

        def body(i, carry):
            base = pl.multiple_of(w_base + i * chunk, 8)
            pltpu.sync_copy(idx_hbm.at[pl.ds(base, chunk)], idx_v)
            pltpu.async_copy(tbl_hbm.at[idx_v], rows_v, sem).wait()
            pltpu.sync_copy(rows_v, out_hbm.at[pl.ds(base, chunk)])
            return carry

        lax.fori_loop(0, n_chunks, body, 0)

    return gather


def _stats_body(g_ref, ps_ref):
    g = g_ref[0]
    mean = jnp.mean(g, axis=1, keepdims=True)
    cen = g - mean
    s = jnp.sum(cen * cen)
    ps_ref[0, 0] = jnp.broadcast_to(s, (128,))


def _final_body(g_ref, nc_ref, ps_ref, a_ref, b_ref, out_ref):
    g = g_ref[0]
    mean = jnp.mean(g, axis=1, keepdims=True)
    cen = g - mean
    tot = jnp.sum(ps_ref[...]) * (1.0 / 128.0)
    denom = jnp.sqrt(tot / jnp.float32(_S * _K * 67 - 1)) + 1e-5
    gp = (cen / denom) * a_ref[...] + b_ref[...]
    rep = nc_ref[0][:, :64]
    out_ref[0, :, :, 0:67] = gp[:, :, 0:67]
    out_ref[0, :, :, 67:131] = jnp.broadcast_to(rep[:, None, :], (_SBLK, _K, 64))


def kernel(xyz, points, affine_alpha, affine_beta):
    B, N, _ = xyz.shape
    D = points.shape[-1]
    nblk = _S // _SBLK

    xr = xyz.transpose(0, 2, 1).reshape(B, 3, N // 1024, 1024)
    fps8 = pl.pallas_call(
        _fps_body,
        grid=(B,),
        in_specs=[pl.BlockSpec((1, 3, N // 1024, 1024), lambda b: (b, 0, 0, 0))],
        out_specs=pl.BlockSpec((1, 8, 128), lambda b: (b, 0, 0)),
        out_shape=jax.ShapeDtypeStruct((B, 8, 128), jnp.int32),
    )(xr)
    fps_idx = fps8.reshape(B, _S)

    tbl = jnp.concatenate(
        [points, xyz, jnp.zeros((B, N, _PAD - D - 3), jnp.float32)], axis=-1)
    tbl_flat = tbl.reshape(B * N, _PAD)
    offs = jnp.arange(B, dtype=jnp.int32) * N

    fps_flat = (fps_idx + offs[:, None]).reshape(B * _S)
    newc = _gather_rows(tbl_flat, fps_flat, B * _S).reshape(B, _S, _PAD)
    new_xyz = newc[:, :, D:D + 3]

    xt = xyz.transpose(0, 2, 1)
    knn_idx = pl.pallas_call(
        _knn_body,
        grid=(B, _S // _SB),
        in_specs=[
            pl.BlockSpec((1, 3, N), lambda b, s: (b, 0, 0)),
            pl.BlockSpec((1, _SB, 3), lambda b, s: (b, s, 0)),
        ],
        out_specs=pl.BlockSpec((1, _SB, _K), lambda b, s: (b, s, 0)),
        out_shape=jax.ShapeDtypeStruct((B, _S, _K), jnp.int32),
        scratch_shapes=[pltpu.VMEM((_SB, N), jnp.float32)],
    )(xt, new_xyz)

    knn_flat = (knn_idx + offs[:, None, None]).reshape(B * _S * _K)
    grouped = _gather_rows(tbl_flat, knn_flat, B * _S * _K).reshape(
        B, _S, _K, _PAD)

    ps = pl.pallas_call(
        _stats_body,
        grid=(B, nblk),
        in_specs=[pl.BlockSpec((1, _SBLK, _K, _PAD), lambda b, j: (b, j, 0, 0))],
        out_specs=pl.BlockSpec((1, 1, 128), lambda b, j: (b * nblk + j, 0, 0)),
        out_shape=jax.ShapeDtypeStruct((B * nblk, 1, 128), jnp.float32),
    )(grouped)

    alpha80 = jnp.pad(affine_alpha.reshape(1, D + 3), ((0, 0), (0, _PAD - D - 3)))
    beta80 = jnp.pad(affine_beta.reshape(1, D + 3), ((0, 0), (0, _PAD - D - 3)))
    out = pl.pallas_call(
        _final_body,
        grid=(B, nblk),
        in_specs=[
            pl.BlockSpec((1, _SBLK, _K, _PAD), lambda b, j: (b, j, 0, 0)),
            pl.BlockSpec((1, _SBLK, _PAD), lambda b, j: (b, j, 0)),
            pl.BlockSpec((nblk, 1, 128), lambda b, j: (b, 0, 0)),
            pl.BlockSpec((1, _PAD), lambda b, j: (0, 0)),
            pl.BlockSpec((1, _PAD), lambda b, j: (0, 0)),
        ],
        out_specs=pl.BlockSpec((1, _SBLK, _K, 2 * D + 3), lambda b, j: (b, j, 0, 0)),
        out_shape=jax.ShapeDtypeStruct((B, _S, _K, 2 * D + 3), jnp.float32),
    )(grouped, newc, ps, alpha80, beta80)

    return (new_xyz, out)


def _gather_rows(tbl_flat, idx_flat, n_rows):
    return _make_sc_gather(n_rows, _PAD)(tbl_flat, idx_flat)


# speedup vs baseline: 1.4711x; 1.4711x over previous
import functools

import jax
import jax.numpy as jnp
from jax import lax
from jax.experimental import pallas as pl
from jax.experimental.pallas import tpu as pltpu
from jax.experimental.pallas import tpu_sc as plsc

_S = 1024
_K = 32
_SB = 128
_SBLK = 128
_PAD = 128


def _fps_body(xr_ref, out_ref):
    b = xr_ref.shape[0]
    xyz = xr_ref[...].reshape(b * 24, 1024)
    rows = lax.broadcasted_iota(jnp.int32, (8, 1024), 0)
    cols = lax.broadcasted_iota(jnp.int32, (8, 1024), 1)
    lin = rows * 1024 + cols
    lin24 = jnp.broadcast_to(lin, (b * 3, 8, 1024)).reshape(b * 24, 1024)
    lin8 = jnp.broadcast_to(lin, (b, 8, 1024)).reshape(b * 8, 1024)
    rows_s = lax.broadcasted_iota(jnp.int32, (8, 128), 0)
    cols_s = lax.broadcasted_iota(jnp.int32, (8, 128), 1)
    lin_s = jnp.broadcast_to(rows_s * 128 + cols_s, (b, 8, 128)).reshape(
        b * 8, 128)
    big_f = jnp.float32(3.0e38)
    big_i = jnp.int32(2 ** 30)

    def body(t, carry):
        last, dist, acc = carry
        last24 = jnp.broadcast_to(last, (b, 24, 1)).reshape(b * 24, 1)
        ps = jnp.sum(jnp.where(lin24 == last24, xyz, 0.0), axis=1, keepdims=True)
        p = jnp.broadcast_to(
            jnp.sum(ps.reshape(b * 3, 8, 1), axis=1, keepdims=True), (b * 3, 8, 1)
        ).reshape(b * 24, 1)
        d24 = (xyz - p) ** 2
        d = ((d24.reshape(b, 3, 8, 1024)[:, 0] + d24.reshape(b, 3, 8, 1024)[:, 1])
             + d24.reshape(b, 3, 8, 1024)[:, 2]).reshape(b * 8, 1024)
        dist = jnp.minimum(dist, d)
        m1 = jnp.max(dist, axis=1, keepdims=True).reshape(b, 8, 1)
        m = jnp.broadcast_to(jnp.max(m1, axis=1, keepdims=True), (b, 8, 1)
                             ).reshape(b * 8, 1)
        cand = jnp.where(dist == m, lin8, big_i)
        n1 = jnp.min(cand, axis=1, keepdims=True).reshape(b, 8, 1)
        nxt = jnp.min(n1, axis=1, keepdims=True)
        nxt8 = jnp.broadcast_to(nxt, (b, 8, 1)).reshape(b * 8, 1)
        acc = jnp.where(lin_s == t, nxt8, acc)
        return nxt, dist, acc

    del big_f
    dist0 = jnp.full((b * 8, 1024), 1e10, jnp.float32)
    acc0 = jnp.zeros((b * 8, 128), jnp.int32)
    _, _, acc = lax.fori_loop(
        1, _S, body, (jnp.zeros((b, 1, 1), jnp.int32), dist0, acc0))
    out_ref[...] = acc.reshape(b, 8, 128)


def _knn_body(xt_ref, q_ref, out_ref, d2_ref):
    xm = xt_ref[0]
    q = q_ref[0]
    n = xm.shape[-1]
    xsq = jnp.sum(xm * xm, axis=0, keepdims=True)
    qsq = jnp.sum(q * q, axis=1, keepdims=True)
    prod = lax.dot_general(q, xm, (((1,), (0,)), ((), ())),
                           preferred_element_type=jnp.float32)
    d2_ref[...] = (qsq - 2.0 * prod) + xsq
    lane = lax.broadcasted_iota(jnp.int32, (_SB, n), 1)
    kcol = lax.broadcasted_iota(jnp.int32, (_SB, _K), 1)
    big = jnp.float32(3.0e38)

    def body(t, _):
        d2 = d2_ref[...]
        m = jnp.min(d2, axis=1, keepdims=True)
        nxt = jnp.min(jnp.where(d2 == m, lane, jnp.int32(2 ** 30)),
                      axis=1, keepdims=True)
        out_ref[0] = jnp.where(kcol == t, nxt, out_ref[0])
        d2_ref[...] = jnp.where(lane == nxt, big, d2)
        return 0

    lax.fori_loop(0, _K, body, 0)


def _make_sc_gather(n_rows, width):
    info = plsc.get_sparse_core_info()
    nc, ns = info.num_cores, info.num_subcores
    nw = nc * ns
    per_w = n_rows // nw
    chunk = 128
    n_chunks = per_w // chunk
    mesh = plsc.VectorSubcoreMesh(core_axis_name="c", subcore_axis_name="s")

    @functools.partial(
        pl.kernel, mesh=mesh,
        out_type=jax.ShapeDtypeStruct((n_rows, width), jnp.float32),
        scratch_types=[
            pltpu.VMEM((chunk,), jnp.int32),
            pltpu.VMEM((chunk, width), jnp.float32),
            pltpu.SemaphoreType.DMA,
        ],
    )
    def gather(tbl_hbm, idx_hbm, out_hbm, idx_v, rows_v, sem):
        wid = lax.axis_index("s") * nc + lax.axis_index("c")
        w_base = wid * per_w

        def body(i, carry):
            base = pl.multiple_of(w_base + i * chunk, 8)
            pltpu.sync_copy(idx_hbm.at[pl.ds(base, chunk)], idx_v)
            pltpu.async_copy(tbl_hbm.at[idx_v], rows_v, sem).wait()
            pltpu.sync_copy(rows_v, out_hbm.at[pl.ds(base, chunk)])
            return carry

        lax.fori_loop(0, n_chunks, body, 0)

    return gather


def _stats_body(g_ref, ps_ref):
    g = g_ref[0]
    mean = jnp.mean(g, axis=1, keepdims=True)
    cen = g - mean
    s = jnp.sum(cen * cen)
    ps_ref[0, 0] = jnp.broadcast_to(s, (128,))


def _final_body(g_ref, nc_ref, ps_ref, a_ref, b_ref, out_ref):
    g = g_ref[0]
    mean = jnp.mean(g, axis=1, keepdims=True)
    cen = g - mean
    tot = jnp.sum(ps_ref[...]) * (1.0 / 128.0)
    denom = jnp.sqrt(tot / jnp.float32(_S * _K * 67 - 1)) + 1e-5
    gp = (cen / denom) * a_ref[...] + b_ref[...]
    rep = nc_ref[0][:, :64]
    out_ref[0, :, :, 0:67] = gp[:, :, 0:67]
    out_ref[0, :, :, 67:131] = jnp.broadcast_to(rep[:, None, :], (_SBLK, _K, 64))


def kernel(xyz, points, affine_alpha, affine_beta):
    B, N, _ = xyz.shape
    D = points.shape[-1]
    nblk = _S // _SBLK

    xr = xyz.transpose(0, 2, 1).reshape(B, 3, N // 1024, 1024)
    fps8 = pl.pallas_call(
        _fps_body,
        grid=(1,),
        in_specs=[pl.BlockSpec((B, 3, N // 1024, 1024), lambda i: (0, 0, 0, 0))],
        out_specs=pl.BlockSpec((B, 8, 128), lambda i: (0, 0, 0)),
        out_shape=jax.ShapeDtypeStruct((B, 8, 128), jnp.int32),
    )(xr)
    fps_idx = fps8.reshape(B, _S)

    tbl = jnp.concatenate(
        [points, xyz, jnp.zeros((B, N, _PAD - D - 3), jnp.float32)], axis=-1)
    tbl_flat = tbl.reshape(B * N, _PAD)
    offs = jnp.arange(B, dtype=jnp.int32) * N

    fps_flat = (fps_idx + offs[:, None]).reshape(B * _S)
    newc = _gather_rows(tbl_flat, fps_flat, B * _S).reshape(B, _S, _PAD)
    new_xyz = newc[:, :, D:D + 3]

    xt = xyz.transpose(0, 2, 1)
    knn_idx = pl.pallas_call(
        _knn_body,
        grid=(B, _S // _SB),
        in_specs=[
            pl.BlockSpec((1, 3, N), lambda b, s: (b, 0, 0)),
            pl.BlockSpec((1, _SB, 3), lambda b, s: (b, s, 0)),
        ],
        out_specs=pl.BlockSpec((1, _SB, _K), lambda b, s: (b, s, 0)),
        out_shape=jax.ShapeDtypeStruct((B, _S, _K), jnp.int32),
        scratch_shapes=[pltpu.VMEM((_SB, N), jnp.float32)],
    )(xt, new_xyz)

    knn_flat = (knn_idx + offs[:, None, None]).reshape(B * _S * _K)
    grouped = _gather_rows(tbl_flat, knn_flat, B * _S * _K).reshape(
        B, _S, _K, _PAD)

    ps = pl.pallas_call(
        _stats_body,
        grid=(B, nblk),
        in_specs=[pl.BlockSpec((1, _SBLK, _K, _PAD), lambda b, j: (b, j, 0, 0))],
        out_specs=pl.BlockSpec((1, 1, 128), lambda b, j: (b * nblk + j, 0, 0)),
        out_shape=jax.ShapeDtypeStruct((B * nblk, 1, 128), jnp.float32),
    )(grouped)

    alpha80 = jnp.pad(affine_alpha.reshape(1, D + 3), ((0, 0), (0, _PAD - D - 3)))
    beta80 = jnp.pad(affine_beta.reshape(1, D + 3), ((0, 0), (0, _PAD - D - 3)))
    out = pl.pallas_call(
        _final_body,
        grid=(B, nblk),
        in_specs=[
            pl.BlockSpec((1, _SBLK, _K, _PAD), lambda b, j: (b, j, 0, 0)),
            pl.BlockSpec((1, _SBLK, _PAD), lambda b, j: (b, j, 0)),
            pl.BlockSpec((nblk, 1, 128), lambda b, j: (b, 0, 0)),
            pl.BlockSpec((1, _PAD), lambda b, j: (0, 0)),
            pl.BlockSpec((1, _PAD), lambda b, j: (0, 0)),
        ],
        out_specs=pl.BlockSpec((1, _SBLK, _K, 2 * D + 3), lambda b, j: (b, j, 0, 0)),
        out_shape=jax.ShapeDtypeStruct((B, _S, _K, 2 * D + 3), jnp.float32),
    )(grouped, newc, ps, alpha80, beta80)

    return (new_xyz, out)


def _gather_rows(tbl_flat, idx_flat, n_rows):
    return _make_sc_gather(n_rows, _PAD)(tbl_flat, idx_flat)


# speedup vs baseline: 1.4944x; 1.0158x over previous
import functools

import jax
import jax.numpy as jnp
from jax import lax
from jax.experimental import pallas as pl
from jax.experimental.pallas import tpu as pltpu
from jax.experimental.pallas import tpu_sc as plsc

_S = 1024
_K = 32
_SB = 128
_SBLK = 128
_PAD = 128


def _fps_body(xr_ref, out_ref):
    b = xr_ref.shape[0]
    xyz = xr_ref[...].reshape(b * 24, 1024)
    rows = lax.broadcasted_iota(jnp.int32, (8, 1024), 0)
    cols = lax.broadcasted_iota(jnp.int32, (8, 1024), 1)
    lin = rows * 1024 + cols
    lin24 = jnp.broadcast_to(lin, (b * 3, 8, 1024)).reshape(b * 24, 1024)
    lin8 = jnp.broadcast_to(lin, (b, 8, 1024)).reshape(b * 8, 1024)
    rows_s = lax.broadcasted_iota(jnp.int32, (8, 128), 0)
    cols_s = lax.broadcasted_iota(jnp.int32, (8, 128), 1)
    lin_s = jnp.broadcast_to(rows_s * 128 + cols_s, (b, 8, 128)).reshape(
        b * 8, 128)
    big_f = jnp.float32(3.0e38)
    big_i = jnp.int32(2 ** 30)

    def body(t, carry):
        last, dist, acc = carry
        last24 = jnp.broadcast_to(last, (b, 24, 1)).reshape(b * 24, 1)
        ps = jnp.sum(jnp.where(lin24 == last24, xyz, 0.0), axis=1, keepdims=True)
        p = jnp.broadcast_to(
            jnp.sum(ps.reshape(b * 3, 8, 1), axis=1, keepdims=True), (b * 3, 8, 1)
        ).reshape(b * 24, 1)
        d24 = (xyz - p) ** 2
        d = ((d24.reshape(b, 3, 8, 1024)[:, 0] + d24.reshape(b, 3, 8, 1024)[:, 1])
             + d24.reshape(b, 3, 8, 1024)[:, 2]).reshape(b * 8, 1024)
        dist = jnp.minimum(dist, d)
        m1 = jnp.max(dist, axis=1, keepdims=True).reshape(b, 8, 1)
        m = jnp.broadcast_to(jnp.max(m1, axis=1, keepdims=True), (b, 8, 1)
                             ).reshape(b * 8, 1)
        cand = jnp.where(dist == m, lin8, big_i)
        n1 = jnp.min(cand, axis=1, keepdims=True).reshape(b, 8, 1)
        nxt = jnp.min(n1, axis=1, keepdims=True)
        nxt8 = jnp.broadcast_to(nxt, (b, 8, 1)).reshape(b * 8, 1)
        acc = jnp.where(lin_s == t, nxt8, acc)
        return nxt, dist, acc

    del big_f
    dist0 = jnp.full((b * 8, 1024), 1e10, jnp.float32)
    acc0 = jnp.zeros((b * 8, 128), jnp.int32)
    _, _, acc = lax.fori_loop(
        1, _S, body, (jnp.zeros((b, 1, 1), jnp.int32), dist0, acc0))
    out_ref[...] = acc.reshape(b, 8, 128)


def _knn_body(xt_ref, q_ref, out_ref, d2_ref, cm_ref):
    xm = xt_ref[0]
    q = q_ref[0]
    n = xm.shape[-1]
    ng = n // 128
    xsq = jnp.sum(xm * xm, axis=0, keepdims=True)
    qsq = jnp.sum(q * q, axis=1, keepdims=True)
    prod = lax.dot_general(q, xm, (((1,), (0,)), ((), ())),
                           preferred_element_type=jnp.float32)
    d2_ref[...] = ((qsq - 2.0 * prod) + xsq).reshape(_SB, ng, 128)
    cm_ref[...] = jnp.min(d2_ref[...], axis=2)

    giota = lax.broadcasted_iota(jnp.int32, (_SB, ng), 1)
    giota3 = lax.broadcasted_iota(jnp.int32, (_SB, ng, 1), 1)
    lane = lax.broadcasted_iota(jnp.int32, (_SB, 128), 1)
    kcol = lax.broadcasted_iota(jnp.int32, (_SB, _K), 1)
    big_f = jnp.float32(3.0e38)
    big_i = jnp.int32(2 ** 30)

    def body(t, carry):
        wm_v, wm_i = carry
        cm = cm_ref[...]
        m1 = jnp.min(cm, axis=1, keepdims=True)
        cstar = jnp.min(jnp.where(cm == m1, giota, big_i),
                        axis=1, keepdims=True)
        ext = jnp.sum(jnp.where(giota3 == cstar[:, :, None], d2_ref[...], 0.0),
                      axis=1)
        absid = cstar * 128 + lane
        taken = (ext < wm_v) | ((ext == wm_v) & (absid <= wm_i))
        exte = jnp.where(taken, big_f, ext)
        lsel = jnp.min(jnp.where(exte == m1, lane, big_i),
                       axis=1, keepdims=True)
        nxt = cstar * 128 + lsel
        out_ref[0] = jnp.where(kcol == t, nxt, out_ref[0])
        newmin = jnp.min(jnp.where(lane == lsel, big_f, exte),
                         axis=1, keepdims=True)
        cm_ref[...] = jnp.where(giota == cstar, newmin, cm)
        return m1, nxt

    lax.fori_loop(0, _K, body,
                  (jnp.full((_SB, 1), -big_f, jnp.float32),
                   jnp.full((_SB, 1), -1, jnp.int32)))


def _make_sc_gather(n_rows, width):
    info = plsc.get_sparse_core_info()
    nc, ns = info.num_cores, info.num_subcores
    nw = nc * ns
    per_w = n_rows // nw
    chunk = 128
    n_chunks = per_w // chunk
    mesh = plsc.VectorSubcoreMesh(core_axis_name="c", subcore_axis_name="s")

    @functools.partial(
        pl.kernel, mesh=mesh,
        out_type=jax.ShapeDtypeStruct((n_rows, width), jnp.float32),
        scratch_types=[
            pltpu.VMEM((chunk,), jnp.int32),
            pltpu.VMEM((chunk, width), jnp.float32),
            pltpu.SemaphoreType.DMA,
        ],
    )
    def gather(tbl_hbm, idx_hbm, out_hbm, idx_v, rows_v, sem):
        wid = lax.axis_index("s") * nc + lax.axis_index("c")
        w_base = wid * per_w

        def body(i, carry):
            base = pl.multiple_of(w_base + i * chunk, 8)
            pltpu.sync_copy(idx_hbm.at[pl.ds(base, chunk)], idx_v)
            pltpu.async_copy(tbl_hbm.at[idx_v], rows_v, sem).wait()
            pltpu.sync_copy(rows_v, out_hbm.at[pl.ds(base, chunk)])
            return carry

        lax.fori_loop(0, n_chunks, body, 0)

    return gather


def _stats_body(g_ref, ps_ref):
    g = g_ref[0]
    mean = jnp.mean(g, axis=1, keepdims=True)
    cen = g - mean
    s = jnp.sum(cen * cen)
    ps_ref[0, 0] = jnp.broadcast_to(s, (128,))


def _final_body(g_ref, nc_ref, ps_ref, a_ref, b_ref, out_ref):
    g = g_ref[0]
    mean = jnp.mean(g, axis=1, keepdims=True)
    cen = g - mean
    tot = jnp.sum(ps_ref[...]) * (1.0 / 128.0)
    denom = jnp.sqrt(tot / jnp.float32(_S * _K * 67 - 1)) + 1e-5
    gp = (cen / denom) * a_ref[...] + b_ref[...]
    rep = nc_ref[0][:, :64]
    out_ref[0, :, :, 0:67] = gp[:, :, 0:67]
    out_ref[0, :, :, 67:131] = jnp.broadcast_to(rep[:, None, :], (_SBLK, _K, 64))


def kernel(xyz, points, affine_alpha, affine_beta):
    B, N, _ = xyz.shape
    D = points.shape[-1]
    nblk = _S // _SBLK

    xr = xyz.transpose(0, 2, 1).reshape(B, 3, N // 1024, 1024)
    fps8 = pl.pallas_call(
        _fps_body,
        grid=(1,),
        in_specs=[pl.BlockSpec((B, 3, N // 1024, 1024), lambda i: (0, 0, 0, 0))],
        out_specs=pl.BlockSpec((B, 8, 128), lambda i: (0, 0, 0)),
        out_shape=jax.ShapeDtypeStruct((B, 8, 128), jnp.int32),
    )(xr)
    fps_idx = fps8.reshape(B, _S)

    tbl = jnp.concatenate(
        [points, xyz, jnp.zeros((B, N, _PAD - D - 3), jnp.float32)], axis=-1)
    tbl_flat = tbl.reshape(B * N, _PAD)
    offs = jnp.arange(B, dtype=jnp.int32) * N

    fps_flat = (fps_idx + offs[:, None]).reshape(B * _S)
    newc = _gather_rows(tbl_flat, fps_flat, B * _S).reshape(B, _S, _PAD)
    new_xyz = newc[:, :, D:D + 3]

    xt = xyz.transpose(0, 2, 1)
    knn_idx = pl.pallas_call(
        _knn_body,
        grid=(B, _S // _SB),
        in_specs=[
            pl.BlockSpec((1, 3, N), lambda b, s: (b, 0, 0)),
            pl.BlockSpec((1, _SB, 3), lambda b, s: (b, s, 0)),
        ],
        out_specs=pl.BlockSpec((1, _SB, _K), lambda b, s: (b, s, 0)),
        out_shape=jax.ShapeDtypeStruct((B, _S, _K), jnp.int32),
        scratch_shapes=[pltpu.VMEM((_SB, N // 128, 128), jnp.float32),
                        pltpu.VMEM((_SB, N // 128), jnp.float32)],
    )(xt, new_xyz)

    knn_flat = (knn_idx + offs[:, None, None]).reshape(B * _S * _K)
    grouped = _gather_rows(tbl_flat, knn_flat, B * _S * _K).reshape(
        B, _S, _K, _PAD)

    ps = pl.pallas_call(
        _stats_body,
        grid=(B, nblk),
        in_specs=[pl.BlockSpec((1, _SBLK, _K, _PAD), lambda b, j: (b, j, 0, 0))],
        out_specs=pl.BlockSpec((1, 1, 128), lambda b, j: (b * nblk + j, 0, 0)),
        out_shape=jax.ShapeDtypeStruct((B * nblk, 1, 128), jnp.float32),
    )(grouped)

    alpha80 = jnp.pad(affine_alpha.reshape(1, D + 3), ((0, 0), (0, _PAD - D - 3)))
    beta80 = jnp.pad(affine_beta.reshape(1, D + 3), ((0, 0), (0, _PAD - D - 3)))
    out = pl.pallas_call(
        _final_body,
        grid=(B, nblk),
        in_specs=[
            pl.BlockSpec((1, _SBLK, _K, _PAD), lambda b, j: (b, j, 0, 0)),
            pl.BlockSpec((1, _SBLK, _PAD), lambda b, j: (b, j, 0)),
            pl.BlockSpec((nblk, 1, 128), lambda b, j: (b, 0, 0)),
            pl.BlockSpec((1, _PAD), lambda b, j: (0, 0)),
            pl.BlockSpec((1, _PAD), lambda b, j: (0, 0)),
        ],
        out_specs=pl.BlockSpec((1, _SBLK, _K, 2 * D + 3), lambda b, j: (b, j, 0, 0)),
        out_shape=jax.ShapeDtypeStruct((B, _S, _K, 2 * D + 3), jnp.float32),
    )(grouped, newc, ps, alpha80, beta80)

    return (new_xyz, out)


def _gather_rows(tbl_flat, idx_flat, n_rows):
    return _make_sc_gather(n_rows, _PAD)(tbl_flat, idx_flat)
